# baseline (speedup 1.0000x reference)
import functools

import jax
import jax.numpy as jnp
from jax import lax
from jax.experimental import pallas as pl
from jax.experimental.pallas import tpu as pltpu
from jax.experimental.pallas import tpu_sc as plsc

EMBED = 128
VOCAB = 21
NC, NS = 2, 16
NW = NC * NS
CHUNK = 128
NBUF = 5


def _sc_lookup(w_flat, b, idx, per_w):
    n_chunks = per_w // CHUNK
    half = n_chunks // NBUF
    mesh = plsc.VectorSubcoreMesh(core_axis_name="c", subcore_axis_name="s")

    @functools.partial(
        pl.kernel,
        out_type=jax.ShapeDtypeStruct((NW * n_chunks, CHUNK, EMBED),
                                      jnp.float32),
        mesh=mesh,
        compiler_params=pltpu.CompilerParams(needs_layout_passes=False),
        scratch_types=[
            pltpu.VMEM((n_chunks, CHUNK), jnp.int32),
            pltpu.VMEM((VOCAB * EMBED,), jnp.float32),
            pltpu.VMEM((EMBED,), jnp.float32),
            pltpu.VMEM((VOCAB, EMBED), jnp.float32),
            pltpu.VMEM((CHUNK, EMBED), jnp.float32),
            pltpu.VMEM((CHUNK, EMBED), jnp.float32),
            pltpu.VMEM((CHUNK, EMBED), jnp.float32),
            pltpu.VMEM((CHUNK, EMBED), jnp.float32),
            pltpu.VMEM((CHUNK, EMBED), jnp.float32),
            pltpu.VMEM_SHARED((VOCAB, EMBED), jnp.float32),
            pltpu.SemaphoreType.DMA,
            pltpu.SemaphoreType.DMA,
            pltpu.SemaphoreType.DMA,
            pltpu.SemaphoreType.DMA,
            pltpu.SemaphoreType.DMA,
            pltpu.SemaphoreType.DMA,
            pltpu.SemaphoreType.DMA,
            pltpu.SemaphoreType.DMA,
            pltpu.SemaphoreType.DMA,
            pltpu.SemaphoreType.DMA,
        ],
    )
    def k(w_hbm, b_hbm, idx_hbm, out_hbm, idx_v, w_v, b_v, tab_v, rows0,
          rows1, rows2, rows3, rows4, shtab, g0, g1, g2, g3, g4,
          s0, s1, s2, s3, s4):
        rows = (rows0, rows1, rows2, rows3, rows4)
        gsem = (g0, g1, g2, g3, g4)
        ssem = (s0, s1, s2, s3, s4)
        iota = lax.iota(jnp.int32, 16)
        sid = lax.axis_index("s")
        wid = sid * NC + lax.axis_index("c")
        gbase = wid * n_chunks
        pltpu.sync_copy(idx_hbm.at[wid], idx_v)
        pltpu.sync_copy(w_hbm, w_v)
        pltpu.sync_copy(b_hbm, b_v)

        bvecs = [b_v[pl.ds(e8 * 16, 16)] for e8 in range(8)]
        for v in range(VOCAB):
            for e8 in range(8):
                widx = (e8 * 16 + iota) * VOCAB + v
                col = plsc.load_gather(w_v, [widx])
                tab_v[v, pl.ds(e8 * 16, 16)] = col + bvecs[e8]

        @pl.when(sid == 0)
        def _():
            pltpu.sync_copy(tab_v, shtab)

        plsc.subcore_barrier()

        def fire(slot, g):
            pltpu.async_copy(shtab.at[idx_v.at[g]], rows[slot], gsem[slot])

        def drain(slot, g):
            pltpu.make_async_copy(shtab.at[idx_v.at[g]], rows[slot],
                                  gsem[slot]).wait()

        def scat(slot, g):
            pltpu.async_copy(rows[slot], out_hbm.at[gbase + g], ssem[slot])

        def scat_wait(slot, g):
            pltpu.make_async_copy(rows[slot], out_hbm.at[gbase + g],
                                  ssem[slot]).wait()

        def body(h, carry):
            g0h = NBUF * h
            for i in range(NBUF):
                @pl.when(h >= 1)
                def _(i=i):
                    scat_wait(i, g0h + i - NBUF)

                fire(i, g0h + i)
            for i in range(NBUF):
                drain(i, g0h + i)
                scat(i, g0h + i)
            return carry

        lax.fori_loop(0, half, body, 0)
        for i in range(NBUF):
            scat_wait(i, NBUF * (half - 1) + i)

    return k(w_flat, b, idx)


def kernel(seq_ids, W, b):
    B, L = seq_ids.shape
    n = B * L
    per_w = n // NW
    idx = seq_ids.reshape(NW, per_w // CHUNK, CHUNK).astype(jnp.int32)
    out = _sc_lookup(W.reshape(-1), b, idx, per_w)
    return out.reshape(B, L, EMBED)
